# baseline (speedup 1.0000x reference)
import functools

import jax
import jax.numpy as jnp
from jax import lax
from jax.experimental import pallas as pl
from jax.experimental.pallas import tpu as pltpu
from jax.experimental.pallas import tpu_sc as plsc


def _sc_gather_rows(table, idx):
    info = plsc.get_sparse_core_info()
    nc, ns = info.num_cores, info.num_subcores
    nw = nc * ns
    b, = idx.shape
    d = table.shape[1]
    b_per_w = b // nw
    mesh = plsc.VectorSubcoreMesh(core_axis_name="c", subcore_axis_name="s")

    @functools.partial(
        pl.kernel, mesh=mesh,
        out_type=jax.ShapeDtypeStruct((b, d), table.dtype),
        scratch_types=[
            pltpu.VMEM((b_per_w,), jnp.int32),
            pltpu.VMEM((b_per_w, d), table.dtype),
            pltpu.SemaphoreType.DMA,
        ],
    )
    def k(table_hbm, idx_hbm, out_hbm, idx_v, rows_v, sem):
        wid = lax.axis_index("s") * nc + lax.axis_index("c")
        base = wid * b_per_w
        pltpu.sync_copy(idx_hbm.at[pl.ds(base, b_per_w)], idx_v)
        pltpu.async_copy(table_hbm.at[idx_v], rows_v, sem).wait()
        pltpu.sync_copy(rows_v, out_hbm.at[pl.ds(base, b_per_w)])

    return k(table, idx)


def _sc_combine_rows(rows_sorted, invperm, n_tokens, k_per_token):
    info = plsc.get_sparse_core_info()
    nc, ns = info.num_cores, info.num_subcores
    nw = nc * ns
    d = rows_sorted.shape[1]
    t_per_w = n_tokens // nw
    rows_per_w = t_per_w * k_per_token
    lanes = info.num_lanes
    mesh = plsc.VectorSubcoreMesh(core_axis_name="c", subcore_axis_name="s")

    @functools.partial(
        pl.kernel, mesh=mesh,
        out_type=jax.ShapeDtypeStruct((n_tokens, d), rows_sorted.dtype),
        scratch_types=[
            pltpu.VMEM((rows_per_w,), jnp.int32),
            pltpu.VMEM((rows_per_w, d), rows_sorted.dtype),
            pltpu.VMEM((t_per_w, d), rows_sorted.dtype),
            pltpu.SemaphoreType.DMA,
        ],
    )
    def k(rows_hbm, inv_hbm, out_hbm, idx_v, rows_v, out_v, sem):
        wid = lax.axis_index("s") * nc + lax.axis_index("c")
        pltpu.sync_copy(inv_hbm.at[pl.ds(wid * rows_per_w, rows_per_w)], idx_v)
        pltpu.async_copy(rows_hbm.at[idx_v], rows_v, sem).wait()

        def body(i, carry):
            for c in range(d // lanes):
                sl = pl.ds(c * lanes, lanes)
                acc = rows_v[i * k_per_token, sl]
                for kk in range(1, k_per_token):
                    acc = acc + rows_v[i * k_per_token + kk, sl]
                out_v[i, sl] = acc
            return carry

        lax.fori_loop(0, t_per_w, body, 0)
        pltpu.sync_copy(out_v, out_hbm.at[pl.ds(wid * t_per_w, t_per_w)])

    return k(rows_sorted, invperm)


_BM = 128


def _grouped_mm_kernel(meta_ref, xs_ref, ws_ref, wg_ref, wu_ref, wd_ref,
                       out_ref):
    t = pl.program_id(0)
    lo = meta_ref[2, t]
    hi = meta_ref[3, t]

    @pl.when(meta_ref[4, t] == 1)
    def _init():
        out_ref[...] = jnp.zeros_like(out_ref)

    @pl.when(hi > lo)
    def _compute():
        rows = lax.broadcasted_iota(jnp.int32, (xs_ref.shape[0], 1), 0)
        mask = (rows >= lo) & (rows < hi)
        xb = jnp.where(mask, xs_ref[...], 0.0)
        g = jnp.dot(xb, wg_ref[0], preferred_element_type=jnp.float32)
        u = jnp.dot(xb, wu_ref[0], preferred_element_type=jnp.float32)
        h = g * lax.logistic(g) * u * ws_ref[...]
        out_ref[...] += jnp.dot(h, wd_ref[0], preferred_element_type=jnp.float32)


def _grouped_mm(xs_sorted, ws_sorted, w_gate, w_up, w_down, meta, n_tiles):
    nk, d = xs_sorted.shape
    inter = w_gate.shape[2]
    bm = _BM
    grid_spec = pltpu.PrefetchScalarGridSpec(
        num_scalar_prefetch=1,
        grid=(n_tiles,),
        in_specs=[
            pl.BlockSpec((bm, d), lambda t, m: (m[0, t], 0)),
            pl.BlockSpec((bm, 1), lambda t, m: (m[0, t], 0)),
            pl.BlockSpec((1, d, inter), lambda t, m: (m[1, t], 0, 0)),
            pl.BlockSpec((1, d, inter), lambda t, m: (m[1, t], 0, 0)),
            pl.BlockSpec((1, inter, d), lambda t, m: (m[1, t], 0, 0)),
        ],
        out_specs=pl.BlockSpec((bm, d), lambda t, m: (m[0, t], 0)),
    )
    return pl.pallas_call(
        _grouped_mm_kernel,
        grid_spec=grid_spec,
        out_shape=jax.ShapeDtypeStruct((nk, d), xs_sorted.dtype),
        compiler_params=pltpu.CompilerParams(
            dimension_semantics=("arbitrary",),
        ),
    )(meta, xs_sorted, ws_sorted, w_gate, w_up, w_down)


def _routing_metadata(expert_indices, expert_weights, n_experts, bm, n_tiles):
    n, k = expert_indices.shape
    nk = n * k
    nb = nk // bm
    i32 = jnp.int32
    flat_e = expert_indices.reshape(-1).astype(i32)
    order = jnp.argsort(flat_e, stable=True).astype(i32)
    sorted_e = flat_e[order]
    token_ids = order // k
    invperm = jnp.argsort(order).astype(i32)
    ws_sorted = expert_weights.reshape(-1)[order].reshape(nk, 1)

    counts = jnp.bincount(flat_e, length=n_experts)
    off = jnp.concatenate(
        [jnp.zeros((1,), i32), jnp.cumsum(counts).astype(i32)])
    first_e = sorted_e[::bm]
    last_e = sorted_e[bm - 1::bm]
    tiles_pb = last_e - first_e + 1
    cum = jnp.cumsum(tiles_pb)
    cumx = cum - tiles_pb
    t_ids = jnp.arange(n_tiles, dtype=i32)
    blk = jnp.searchsorted(cum, t_ids, side="right").astype(i32)
    blk_c = jnp.minimum(blk, nb - 1)
    e_t = jnp.clip(first_e[blk_c] + (t_ids - cumx[blk_c]), 0, n_experts - 1)
    valid = t_ids < cum[-1]
    lo = jnp.clip(jnp.maximum(off[e_t], blk_c * bm) - blk_c * bm, 0, bm)
    hi = jnp.clip(jnp.minimum(off[e_t + 1], (blk_c + 1) * bm) - blk_c * bm,
                  0, bm)
    lo = jnp.where(valid, lo, 0)
    hi = jnp.where(valid, hi, 0)
    first = ((t_ids == cumx[blk_c]) & valid).astype(i32)
    meta = jnp.stack([blk_c, e_t, lo, hi, first])
    return token_ids, invperm, ws_sorted, meta


def kernel(x, expert_weights, w_gate, w_up, w_down, expert_indices, top_k):
    n, d = x.shape
    e_num = w_gate.shape[0]
    k = expert_indices.shape[1]
    nk = n * k
    bm = _BM
    n_tiles = nk // bm + e_num - 1

    token_ids, invperm, ws_sorted, meta = _routing_metadata(
        expert_indices, expert_weights, e_num, bm, n_tiles)

    xs_sorted = _sc_gather_rows(x, token_ids)
    down_sorted = _grouped_mm(
        xs_sorted, ws_sorted.astype(x.dtype), w_gate, w_up, w_down, meta,
        n_tiles)
    return _sc_combine_rows(down_sorted, invperm, n, k)


# speedup vs baseline: 1.1857x; 1.1857x over previous
import functools

import jax
import jax.numpy as jnp
from jax import lax
from jax.experimental import pallas as pl
from jax.experimental.pallas import tpu as pltpu
from jax.experimental.pallas import tpu_sc as plsc


def _sc_scatter_rows(x, s2):
    info = plsc.get_sparse_core_info()
    nc, ns = info.num_cores, info.num_subcores
    nw = nc * ns
    n, d = x.shape
    kk = s2.shape[0]
    t_per_w = n // nw
    mesh = plsc.VectorSubcoreMesh(core_axis_name="c", subcore_axis_name="s")

    @functools.partial(
        pl.kernel, mesh=mesh,
        out_type=jax.ShapeDtypeStruct((n * kk, d), x.dtype),
        scratch_types=[
            pltpu.VMEM((kk, t_per_w), jnp.int32),
            pltpu.VMEM((t_per_w, d), x.dtype),
            pltpu.SemaphoreType.DMA,
        ],
    )
    def k(x_hbm, s2_hbm, out_hbm, idx_v, rows_v, sem):
        wid = lax.axis_index("s") * nc + lax.axis_index("c")
        base = wid * t_per_w
        for j in range(kk):
            pltpu.sync_copy(s2_hbm.at[j, pl.ds(base, t_per_w)], idx_v.at[j])
        pltpu.sync_copy(x_hbm.at[pl.ds(base, t_per_w)], rows_v)
        copies = [
            pltpu.async_copy(rows_v, out_hbm.at[idx_v.at[j]], sem)
            for j in range(kk)
        ]
        for c in copies:
            c.wait()

    return k(x, s2)


def _sc_combine_rows(rows_sorted, invperm, n_tokens, k_per_token):
    info = plsc.get_sparse_core_info()
    nc, ns = info.num_cores, info.num_subcores
    nw = nc * ns
    d = rows_sorted.shape[1]
    t_per_w = n_tokens // nw
    rows_per_w = t_per_w * k_per_token
    lanes = info.num_lanes
    mesh = plsc.VectorSubcoreMesh(core_axis_name="c", subcore_axis_name="s")

    @functools.partial(
        pl.kernel, mesh=mesh,
        out_type=jax.ShapeDtypeStruct((n_tokens, d), rows_sorted.dtype),
        scratch_types=[
            pltpu.VMEM((rows_per_w,), jnp.int32),
            pltpu.VMEM((rows_per_w, d), rows_sorted.dtype),
            pltpu.VMEM((t_per_w, d), rows_sorted.dtype),
            pltpu.SemaphoreType.DMA,
        ],
    )
    def k(rows_hbm, inv_hbm, out_hbm, idx_v, rows_v, out_v, sem):
        wid = lax.axis_index("s") * nc + lax.axis_index("c")
        pltpu.sync_copy(inv_hbm.at[pl.ds(wid * rows_per_w, rows_per_w)], idx_v)
        pltpu.async_copy(rows_hbm.at[idx_v], rows_v, sem).wait()

        def body(i, carry):
            for c in range(d // lanes):
                sl = pl.ds(c * lanes, lanes)
                acc = rows_v[i * k_per_token, sl]
                for j in range(1, k_per_token):
                    acc = acc + rows_v[i * k_per_token + j, sl]
                out_v[i, sl] = acc
            return carry

        lax.fori_loop(0, t_per_w, body, 0)
        pltpu.sync_copy(out_v, out_hbm.at[pl.ds(wid * t_per_w, t_per_w)])

    return k(rows_sorted, invperm)


_BM = 256


def _grouped_mm_kernel(meta_ref, xs_ref, ws_ref, wg_ref, wu_ref, wd_ref,
                       out_ref):
    t = pl.program_id(0)
    lo = meta_ref[2, t]
    hi = meta_ref[3, t]

    @pl.when(meta_ref[4, t] == 1)
    def _init():
        out_ref[...] = jnp.zeros_like(out_ref)

    @pl.when(hi > lo)
    def _compute():
        rows = lax.broadcasted_iota(jnp.int32, (xs_ref.shape[0], 1), 0)
        mask = (rows >= lo) & (rows < hi)
        xb = jnp.where(mask, xs_ref[...], 0.0)
        g = jnp.dot(xb, wg_ref[0], preferred_element_type=jnp.float32)
        u = jnp.dot(xb, wu_ref[0], preferred_element_type=jnp.float32)
        h = g * lax.logistic(g) * u * ws_ref[...]
        out_ref[...] += jnp.dot(h, wd_ref[0], preferred_element_type=jnp.float32)


def _grouped_mm(xs_sorted, ws_sorted, w_gate, w_up, w_down, meta, n_tiles):
    nk, d = xs_sorted.shape
    inter = w_gate.shape[2]
    bm = _BM
    grid_spec = pltpu.PrefetchScalarGridSpec(
        num_scalar_prefetch=1,
        grid=(n_tiles,),
        in_specs=[
            pl.BlockSpec((bm, d), lambda t, m: (m[0, t], 0)),
            pl.BlockSpec((bm, 1), lambda t, m: (m[0, t], 0)),
            pl.BlockSpec((1, d, inter), lambda t, m: (m[1, t], 0, 0)),
            pl.BlockSpec((1, d, inter), lambda t, m: (m[1, t], 0, 0)),
            pl.BlockSpec((1, inter, d), lambda t, m: (m[1, t], 0, 0)),
        ],
        out_specs=pl.BlockSpec((bm, d), lambda t, m: (m[0, t], 0)),
    )
    return pl.pallas_call(
        _grouped_mm_kernel,
        grid_spec=grid_spec,
        out_shape=jax.ShapeDtypeStruct((nk, d), xs_sorted.dtype),
        compiler_params=pltpu.CompilerParams(
            dimension_semantics=("arbitrary",),
        ),
    )(meta, xs_sorted, ws_sorted, w_gate, w_up, w_down)


def _routing_metadata(expert_indices, expert_weights, n_experts, bm, n_tiles):
    n, k = expert_indices.shape
    nk = n * k
    nb = nk // bm
    i32 = jnp.int32
    flat_e = expert_indices.reshape(-1).astype(i32)
    e_ids = jnp.arange(n_experts, dtype=i32)
    oh = (flat_e[:, None] == e_ids[None, :]).astype(i32)
    csum = jnp.cumsum(oh, axis=0)
    counts = csum[-1]
    cum_counts = jnp.cumsum(counts)
    off9 = jnp.concatenate([jnp.zeros((1,), i32),
                            cum_counts.astype(i32)])
    within = jnp.sum((csum - oh) * oh, axis=1)
    base = jnp.sum(off9[None, :n_experts] * oh, axis=1)
    s = (base + within).astype(i32)

    ws_flat = expert_weights.reshape(-1).astype(expert_weights.dtype)
    ws_sorted = jnp.zeros((nk,), ws_flat.dtype).at[s].set(ws_flat)

    first_pos = jnp.arange(nb, dtype=i32) * bm
    first_e = jnp.sum((cum_counts[None, :] <= first_pos[:, None]), axis=1)
    last_e = jnp.sum((cum_counts[None, :] <= first_pos[:, None] + (bm - 1)),
                     axis=1)
    tiles_pb = (last_e - first_e + 1).astype(i32)
    cum = jnp.cumsum(tiles_pb)
    cumx = cum - tiles_pb
    t_ids = jnp.arange(n_tiles, dtype=i32)
    blk = jnp.sum((cum[None, :] <= t_ids[:, None]), axis=1).astype(i32)
    blk_c = jnp.minimum(blk, nb - 1)
    oh_blk = (jnp.arange(nb, dtype=i32)[None, :] == blk_c[:, None])
    first_e_t = jnp.sum(jnp.where(oh_blk, first_e[None, :], 0), axis=1)
    cumx_t = jnp.sum(jnp.where(oh_blk, cumx[None, :], 0), axis=1)
    e_t = jnp.clip(first_e_t + (t_ids - cumx_t), 0, n_experts - 1)
    oh_e = (jnp.arange(n_experts + 1, dtype=i32)[None, :] == e_t[:, None])
    oh_e1 = (jnp.arange(n_experts + 1, dtype=i32)[None, :] ==
             (e_t + 1)[:, None])
    off_e = jnp.sum(jnp.where(oh_e, off9[None, :], 0), axis=1)
    off_e1 = jnp.sum(jnp.where(oh_e1, off9[None, :], 0), axis=1)
    valid = t_ids < cum[-1]
    lo = jnp.clip(jnp.maximum(off_e, blk_c * bm) - blk_c * bm, 0, bm)
    hi = jnp.clip(jnp.minimum(off_e1, (blk_c + 1) * bm) - blk_c * bm, 0, bm)
    lo = jnp.where(valid, lo, 0)
    hi = jnp.where(valid, hi, 0)
    first = ((t_ids == cumx_t) & valid).astype(i32)
    meta = jnp.stack([blk_c.astype(i32), e_t.astype(i32), lo.astype(i32),
                      hi.astype(i32), first])
    return s, ws_sorted.reshape(nk, 1), meta


def kernel(x, expert_weights, w_gate, w_up, w_down, expert_indices, top_k):
    n, d = x.shape
    e_num = w_gate.shape[0]
    k = expert_indices.shape[1]
    nk = n * k
    bm = _BM
    n_tiles = nk // bm + e_num - 1

    s, ws_sorted, meta = _routing_metadata(
        expert_indices, expert_weights, e_num, bm, n_tiles)

    s2 = s.reshape(n, k).T
    xs_sorted = _sc_scatter_rows(x, s2)
    down_sorted = _grouped_mm(
        xs_sorted, ws_sorted.astype(x.dtype), w_gate, w_up, w_down, meta,
        n_tiles)
    return _sc_combine_rows(down_sorted, s, n, k)


# speedup vs baseline: 1.2776x; 1.0775x over previous
import functools

import jax
import jax.numpy as jnp
from jax import lax
from jax.experimental import pallas as pl
from jax.experimental.pallas import tpu as pltpu
from jax.experimental.pallas import tpu_sc as plsc


def _sc_scatter_rows(x, s2, out_rows):
    info = plsc.get_sparse_core_info()
    nc, ns = info.num_cores, info.num_subcores
    nw = nc * ns
    n, d = x.shape
    kk = s2.shape[0]
    t_per_w = n // nw
    mesh = plsc.VectorSubcoreMesh(core_axis_name="c", subcore_axis_name="s")

    @functools.partial(
        pl.kernel, mesh=mesh,
        out_type=jax.ShapeDtypeStruct((out_rows, d), x.dtype),
        scratch_types=[
            pltpu.VMEM((kk, t_per_w), jnp.int32),
            pltpu.VMEM((t_per_w, d), x.dtype),
            pltpu.SemaphoreType.DMA,
        ],
    )
    def k(x_hbm, s2_hbm, out_hbm, idx_v, rows_v, sem):
        wid = lax.axis_index("s") * nc + lax.axis_index("c")
        base = wid * t_per_w
        for j in range(kk):
            pltpu.sync_copy(s2_hbm.at[j, pl.ds(base, t_per_w)], idx_v.at[j])
        pltpu.sync_copy(x_hbm.at[pl.ds(base, t_per_w)], rows_v)
        copies = [
            pltpu.async_copy(rows_v, out_hbm.at[idx_v.at[j]], sem)
            for j in range(kk)
        ]
        for c in copies:
            c.wait()

    return k(x, s2)


def _sc_combine_rows(rows_sorted, invperm, n_tokens, k_per_token):
    info = plsc.get_sparse_core_info()
    nc, ns = info.num_cores, info.num_subcores
    nw = nc * ns
    d = rows_sorted.shape[1]
    t_per_w = n_tokens // nw
    rows_per_w = t_per_w * k_per_token
    lanes = info.num_lanes
    mesh = plsc.VectorSubcoreMesh(core_axis_name="c", subcore_axis_name="s")

    @functools.partial(
        pl.kernel, mesh=mesh,
        out_type=jax.ShapeDtypeStruct((n_tokens, d), rows_sorted.dtype),
        scratch_types=[
            pltpu.VMEM((rows_per_w,), jnp.int32),
            pltpu.VMEM((rows_per_w, d), rows_sorted.dtype),
            pltpu.VMEM((t_per_w, d), rows_sorted.dtype),
            pltpu.SemaphoreType.DMA,
        ],
    )
    def k(rows_hbm, inv_hbm, out_hbm, idx_v, rows_v, out_v, sem):
        wid = lax.axis_index("s") * nc + lax.axis_index("c")
        pltpu.sync_copy(inv_hbm.at[pl.ds(wid * rows_per_w, rows_per_w)], idx_v)
        pltpu.async_copy(rows_hbm.at[idx_v], rows_v, sem).wait()

        def body(i, carry):
            for c in range(d // lanes):
                sl = pl.ds(c * lanes, lanes)
                acc = rows_v[i * k_per_token, sl]
                for j in range(1, k_per_token):
                    acc = acc + rows_v[i * k_per_token + j, sl]
                out_v[i, sl] = acc
            return carry

        lax.fori_loop(0, t_per_w, body, 0)
        pltpu.sync_copy(out_v, out_hbm.at[pl.ds(wid * t_per_w, t_per_w)])

    return k(rows_sorted, invperm)


_BM = 256


def _grouped_mm_kernel(meta_ref, xs_ref, ws_ref, wg_ref, wu_ref, wd_ref,
                       out_ref):
    t = pl.program_id(0)
    hi = meta_ref[2, t]

    @pl.when(hi > 0)
    def _compute():
        rows = lax.broadcasted_iota(jnp.int32, (xs_ref.shape[0], 1), 0)
        xb = jnp.where(rows < hi, xs_ref[...], 0.0)
        g = jnp.dot(xb, wg_ref[0], preferred_element_type=jnp.float32)
        u = jnp.dot(xb, wu_ref[0], preferred_element_type=jnp.float32)
        h = g * lax.logistic(g) * u * ws_ref[...]
        out_ref[...] = jnp.dot(h, wd_ref[0], preferred_element_type=jnp.float32)


def _grouped_mm(xs_sorted, ws_sorted, w_gate, w_up, w_down, meta, n_tiles):
    nk, d = xs_sorted.shape
    inter = w_gate.shape[2]
    bm = _BM
    grid_spec = pltpu.PrefetchScalarGridSpec(
        num_scalar_prefetch=1,
        grid=(n_tiles,),
        in_specs=[
            pl.BlockSpec((bm, d), lambda t, m: (m[0, t], 0)),
            pl.BlockSpec((bm, 1), lambda t, m: (m[0, t], 0)),
            pl.BlockSpec((1, d, inter), lambda t, m: (m[1, t], 0, 0)),
            pl.BlockSpec((1, d, inter), lambda t, m: (m[1, t], 0, 0)),
            pl.BlockSpec((1, inter, d), lambda t, m: (m[1, t], 0, 0)),
        ],
        out_specs=pl.BlockSpec((bm, d), lambda t, m: (m[0, t], 0)),
    )
    return pl.pallas_call(
        _grouped_mm_kernel,
        grid_spec=grid_spec,
        out_shape=jax.ShapeDtypeStruct((nk, d), xs_sorted.dtype),
        compiler_params=pltpu.CompilerParams(
            dimension_semantics=("arbitrary",),
        ),
    )(meta, xs_sorted, ws_sorted, w_gate, w_up, w_down)


def _routing_metadata(expert_indices, expert_weights, n_experts, bm, n_tiles):
    n, k = expert_indices.shape
    nk = n * k
    nkp = n_tiles * bm
    i32 = jnp.int32
    flat_e = expert_indices.reshape(-1).astype(i32)
    e_ids = jnp.arange(n_experts, dtype=i32)
    oh = (flat_e[:, None] == e_ids[None, :]).astype(i32)
    csum = jnp.cumsum(oh, axis=0)
    counts = csum[-1]
    tiles_e = (counts + bm - 1) // bm
    pad_e = tiles_e * bm
    off_p = jnp.concatenate([jnp.zeros((1,), i32),
                             jnp.cumsum(pad_e).astype(i32)])
    within = jnp.sum((csum - oh) * oh, axis=1)
    base = jnp.sum(off_p[None, :n_experts] * oh, axis=1)
    s = (base + within).astype(i32)

    ws_flat = expert_weights.reshape(-1).astype(expert_weights.dtype)
    ws_sorted = jnp.zeros((nkp,), ws_flat.dtype).at[s].set(ws_flat)

    cum_tiles = jnp.cumsum(tiles_e).astype(i32)
    total_tiles = cum_tiles[-1]
    t_ids = jnp.arange(n_tiles, dtype=i32)
    tc = jnp.minimum(t_ids, total_tiles - 1)
    e_t = jnp.sum((cum_tiles[None, :] <= tc[:, None]), axis=1).astype(i32)
    oh_e = (e_ids[None, :] == e_t[:, None])
    tiles_t = jnp.sum(jnp.where(oh_e, tiles_e[None, :], 0), axis=1)
    cum_t = jnp.sum(jnp.where(oh_e, cum_tiles[None, :], 0), axis=1)
    offp_t = jnp.sum(jnp.where(oh_e, off_p[None, :n_experts], 0), axis=1)
    cnt_t = jnp.sum(jnp.where(oh_e, counts[None, :], 0), axis=1)
    j = tc - (cum_t - tiles_t)
    blk = offp_t // bm + j
    hi = jnp.clip(cnt_t - j * bm, 0, bm)
    hi = jnp.where(t_ids < total_tiles, hi, 0)
    meta = jnp.stack([blk.astype(i32), e_t, hi.astype(i32)])
    return s, ws_sorted.reshape(nkp, 1), meta


def kernel(x, expert_weights, w_gate, w_up, w_down, expert_indices, top_k):
    n, d = x.shape
    e_num = w_gate.shape[0]
    k = expert_indices.shape[1]
    nk = n * k
    bm = _BM
    n_tiles = nk // bm + e_num - 1

    s, ws_sorted, meta = _routing_metadata(
        expert_indices, expert_weights, e_num, bm, n_tiles)

    s2 = s.reshape(n, k).T
    xs_sorted = _sc_scatter_rows(x, s2, n_tiles * bm)
    down_sorted = _grouped_mm(
        xs_sorted, ws_sorted.astype(x.dtype), w_gate, w_up, w_down, meta,
        n_tiles)
    return _sc_combine_rows(down_sorted, s, n, k)
